# Initial kernel scaffold; baseline (speedup 1.0000x reference)
import jax
import jax.numpy as jnp
from jax.experimental import pallas as pl


def kernel(x_low, x_high, z_std, ei_l2h, ei_hh, ds_Wl, ds_Wr, ds_att, ds_b, bn0_g, bn0_b, g1_Wl, g1_Wr, g1_att, g1_b, bn1_g, bn1_b, g2_Wl, g2_Wr, g2_att, g2_b, bn2_g, bn2_b, g3_Wl, g3_Wr, g3_att, g3_b, bn3_g, bn3_b, g4_Wl, g4_Wr, g4_att, g4_b, bn4_g, bn4_b, g5_Wl, g5_Wr, g5_att, g5_b, pred_W1, pred_b1, pred_W2, pred_b2, pred_W3, pred_b3):
    raise NotImplementedError("write your pallas kernel here")


# speedup vs baseline: 1.1439x; 1.1439x over previous
import functools
import jax
import jax.numpy as jnp
from jax.experimental import pallas as pl


def _gat_nomax(x_src, x_dst, ei, Wl, Wr, att, b, heads, out_ch, self_loops):
    N = x_dst.shape[0]
    if self_loops:
        loop = jnp.arange(N, dtype=ei.dtype)
        ei = jnp.concatenate([ei, jnp.stack([loop, loop])], axis=1)
    src, dst = ei[0], ei[1]
    hl = (x_src @ Wl).reshape(x_src.shape[0], heads, out_ch)
    hr = (x_dst @ Wr).reshape(N, heads, out_ch)
    e = jax.nn.leaky_relu(hl[src] + hr[dst], 0.2)
    logits = jnp.sum(e * att[None, :, :], axis=-1)
    w = jnp.exp(logits)
    denom = jax.ops.segment_sum(w, dst, num_segments=N)
    msg = hl[src] * w[:, :, None]
    out = jax.ops.segment_sum(msg, dst, num_segments=N)
    out = out / jnp.clip(denom, 1e-16, None)[:, :, None]
    deg = jax.ops.segment_sum(jnp.ones((dst.shape[0],), dtype=out.dtype), dst, num_segments=N)
    out = out / jnp.clip(deg, 1.0, None)[:, None, None]
    return out.reshape(N, heads * out_ch) + b


def _bn(x, g, b):
    m = jnp.mean(x, axis=0)
    v = jnp.var(x, axis=0)
    return (x - m) / jnp.sqrt(v + 1e-5) * g + b


def _mlp_body(h_ref, w1_ref, b1_ref, w2_ref, b2_ref, w3_ref, b3_ref, o_ref):
    h = h_ref[...]
    h = jnp.maximum(h, 0.0)
    h = jnp.maximum(h @ w1_ref[...] + b1_ref[...], 0.0)
    h = jnp.maximum(h @ w2_ref[...] + b2_ref[...], 0.0)
    o_ref[...] = h @ w3_ref[...] + b3_ref[...]


def _pred_mlp(h, W1, b1, W2, b2, W3, b3):
    N = h.shape[0]
    BLK = 2000
    grid = (N // BLK,)
    full = lambda s: pl.BlockSpec(s, lambda i: (0,) * len(s))
    return pl.pallas_call(
        _mlp_body,
        grid=grid,
        in_specs=[
            pl.BlockSpec((BLK, 64), lambda i: (i, 0)),
            full((64, 64)), full((64,)), full((64, 32)), full((32,)),
            full((32, 1)), full((1,)),
        ],
        out_specs=pl.BlockSpec((BLK, 1), lambda i: (i, 0)),
        out_shape=jax.ShapeDtypeStruct((N, 1), jnp.float32),
    )(h, W1, b1, W2, b2, W3, b3)


def kernel(x_low, x_high, z_std, ei_l2h, ei_hh, ds_Wl, ds_Wr, ds_att, ds_b, bn0_g, bn0_b, g1_Wl, g1_Wr, g1_att, g1_b, bn1_g, bn1_b, g2_Wl, g2_Wr, g2_att, g2_b, bn2_g, bn2_b, g3_Wl, g3_Wr, g3_att, g3_b, bn3_g, bn3_b, g4_Wl, g4_Wr, g4_att, g4_b, bn4_g, bn4_b, g5_Wl, g5_Wr, g5_att, g5_b, pred_W1, pred_b1, pred_W2, pred_b2, pred_W3, pred_b3):
    h = _gat_nomax(x_low, x_high, ei_l2h, ds_Wl, ds_Wr, ds_att, ds_b, 1, 64, False)
    h = jnp.concatenate([z_std, h], axis=-1)
    h = _bn(h, bn0_g, bn0_b)
    h = _gat_nomax(h, h, ei_hh, g1_Wl, g1_Wr, g1_att, g1_b, 2, 64, True)
    h = jax.nn.relu(_bn(h, bn1_g, bn1_b))
    h = _gat_nomax(h, h, ei_hh, g2_Wl, g2_Wr, g2_att, g2_b, 2, 64, True)
    h = jax.nn.relu(_bn(h, bn2_g, bn2_b))
    h = _gat_nomax(h, h, ei_hh, g3_Wl, g3_Wr, g3_att, g3_b, 2, 64, True)
    h = jax.nn.relu(_bn(h, bn3_g, bn3_b))
    h = _gat_nomax(h, h, ei_hh, g4_Wl, g4_Wr, g4_att, g4_b, 2, 64, True)
    h = jax.nn.relu(_bn(h, bn4_g, bn4_b))
    h = _gat_nomax(h, h, ei_hh, g5_Wl, g5_Wr, g5_att, g5_b, 1, 64, True)
    return _pred_mlp(h, pred_W1, pred_b1, pred_W2, pred_b2, pred_W3, pred_b3)


# speedup vs baseline: 2.6758x; 2.3392x over previous
import functools
import jax
import jax.numpy as jnp
from jax import lax
from jax.experimental import pallas as pl
from jax.experimental.pallas import tpu as pltpu, tpu_sc as plsc

NB = 128
R = 392
NTH = 51200
NTL = 10240
NREAL = 50000
BLK = 1024


def _make_sc_edge(G, H):
    W = G * 16 + 16
    GH = G // H
    BPW = NB // 32

    @functools.partial(
        pl.kernel,
        mesh=plsc.VectorSubcoreMesh(core_axis_name="c", subcore_axis_name="s"),
        compiler_params=pltpu.CompilerParams(needs_layout_passes=False,
                                             use_tc_tiling_on_sc=False),
        out_type=(jax.ShapeDtypeStruct((NB * R, W), jnp.float32),),
        scratch_types=[
            pltpu.VMEM((16,), jnp.int32),
            pltpu.VMEM((16,), jnp.int32),
            pltpu.VMEM((16, G * 16), jnp.float32),
            pltpu.VMEM((16, G * 16), jnp.float32),
            pltpu.VMEM((G * 16, 16), jnp.float32),
            pltpu.VMEM((R + 1, W), jnp.float32),
            pltpu.VMEM((NB + 16,), jnp.int32),
            pltpu.VMEM((G * 16,), jnp.float32),
            pltpu.SemaphoreType.DMA,
        ],
    )
    def sc_edge(hl_hbm, hr_hbm, src_hbm, dst_hbm, poff_hbm, att_hbm, zeros_hbm,
                out, sidx, didx, hlb, hrb, tbuf, acc, poffv, attv, sem):
        wid = lax.axis_index("s") * 2 + lax.axis_index("c")
        pltpu.sync_copy(poff_hbm, poffv)
        pltpu.sync_copy(att_hbm, attv)
        iota = lax.iota(jnp.int32, 16)
        avs = [attv[pl.ds(g * 16, 16)] for g in range(G)]
        for j in range(BPW):
            bidx = wid * BPW + j
            base = pl.multiple_of(bidx * R, 8)
            pltpu.sync_copy(zeros_hbm, acc)
            bv = jnp.full((16,), 0, jnp.int32) + bidx
            start = plsc.load_gather(poffv, [bv])[0]
            end = plsc.load_gather(poffv, [bv + 1])[0]
            nch = (end - start) // 16

            def chunk(ci, carry):
                off = pl.multiple_of(start + ci * 16, 16)
                pltpu.sync_copy(src_hbm.at[pl.ds(off, 16)], sidx)
                pltpu.sync_copy(dst_hbm.at[pl.ds(off, 16)], didx)
                pltpu.async_copy(hl_hbm.at[sidx], hlb, sem).wait()
                pltpu.async_copy(hr_hbm.at[didx], hrb, sem).wait()
                ldst = didx[...] - base
                ss = [jnp.zeros((16,), jnp.float32) for _ in range(H)]
                for g in range(G):
                    av = avs[g]
                    hh = g // GH
                    for k in range(16):
                        cidx = g * 16 + k
                        cv = jnp.full((16,), cidx, jnp.int32)
                        colL = plsc.load_gather(hlb, [iota, cv])
                        colR = plsc.load_gather(hrb, [iota, cv])
                        tbuf[cidx, :] = colL
                        e = colL + colR
                        e = jnp.where(e > 0, e, e * jnp.float32(0.2))
                        ss[hh] = ss[hh] + av[k] * e
                ws = [jnp.exp(s) for s in ss]
                for g in range(G):
                    wv = ws[g // GH]
                    for k in range(16):
                        cidx = g * 16 + k
                        cv = jnp.full((16,), cidx, jnp.int32)
                        plsc.addupdate_scatter(acc, [ldst, cv], wv * tbuf[cidx, :])
                plsc.addupdate_scatter(
                    acc, [ldst, jnp.full((16,), G * 16, jnp.int32)], ws[0])
                if H == 2:
                    plsc.addupdate_scatter(
                        acc, [ldst, jnp.full((16,), G * 16 + 1, jnp.int32)], ws[1])
                plsc.addupdate_scatter(
                    acc, [ldst, jnp.full((16,), G * 16 + 2, jnp.int32)],
                    jnp.ones((16,), jnp.float32))
                return carry

            lax.fori_loop(0, nch, chunk, jnp.int32(0))
            pltpu.sync_copy(acc.at[pl.ds(0, R)], out.at[pl.ds(base, R)])

    return sc_edge


_sc41 = _make_sc_edge(4, 1)
_sc82 = _make_sc_edge(8, 2)


def _prep_edges(src, dst):
    E = src.shape[0]
    src = src.astype(jnp.int32)
    dst = dst.astype(jnp.int32)
    order = jnp.argsort(dst)
    s = src[order]
    d = dst[order]
    b = d // R
    cnt = jax.ops.segment_sum(jnp.ones((E,), jnp.int32), b, num_segments=NB)
    pcnt = ((cnt + 15) // 16) * 16
    z1 = jnp.zeros((1,), jnp.int32)
    poff = jnp.concatenate([z1, jnp.cumsum(pcnt)])
    cstart = jnp.concatenate([z1, jnp.cumsum(cnt)])
    pos = poff[b] + (jnp.arange(E, dtype=jnp.int32) - cstart[b])
    Epad = E + 16 * NB
    slots = jnp.arange(Epad, dtype=jnp.int32)
    slot_b = jnp.minimum(jnp.searchsorted(poff[1:], slots, side='right'), NB - 1)
    srcp = jnp.zeros((Epad,), jnp.int32).at[pos].set(s)
    dstp = ((slot_b + 1) * R).astype(jnp.int32).at[pos].set(d)
    poff_pad = jnp.concatenate([poff, jnp.full((15,), poff[-1], jnp.int32)])
    return srcp, dstp, poff_pad


def _mm1_body(x_ref, w_ref, o_ref):
    o_ref[...] = x_ref[...] @ w_ref[...]


def _mm1(x, w):
    N, K = x.shape
    M = w.shape[1]
    return pl.pallas_call(
        _mm1_body,
        grid=(N // BLK,),
        in_specs=[pl.BlockSpec((BLK, K), lambda i: (i, 0)),
                  pl.BlockSpec((K, M), lambda i: (0, 0))],
        out_specs=pl.BlockSpec((BLK, M), lambda i: (i, 0)),
        out_shape=jax.ShapeDtypeStruct((N, M), jnp.float32),
    )(x, w)


def _norm_rows(s, H, C):
    num = s[:, :C]
    if H == 1:
        den = jnp.broadcast_to(s[:, C][:, None], num.shape)
    else:
        hc = C // 2
        den = jnp.concatenate(
            [jnp.broadcast_to(s[:, C][:, None], (s.shape[0], hc)),
             jnp.broadcast_to(s[:, C + 1][:, None], (s.shape[0], hc))], axis=1)
    deg = jnp.clip(s[:, C + 2], 1.0, None)
    return num / jnp.clip(den, 1e-16, None) / deg[:, None]


def _a_body(s_ref, b_ref, o_ref, *, H, C):
    o_ref[...] = _norm_rows(s_ref[...], H, C) + b_ref[...]


def _a_body_z(s_ref, b_ref, z_ref, o_ref, *, H, C):
    h = _norm_rows(s_ref[...], H, C) + b_ref[...]
    o_ref[...] = jnp.concatenate([z_ref[...], h], axis=1)


def _a(scout, bias, H, z=None):
    C = bias.shape[0]
    W = scout.shape[1]
    N = scout.shape[0]
    if z is None:
        body = functools.partial(_a_body, H=H, C=C)
        in_specs = [pl.BlockSpec((BLK, W), lambda i: (i, 0)),
                    pl.BlockSpec((C,), lambda i: (0,))]
        args = (scout, bias)
        Cout = C
    else:
        body = functools.partial(_a_body_z, H=H, C=C)
        in_specs = [pl.BlockSpec((BLK, W), lambda i: (i, 0)),
                    pl.BlockSpec((C,), lambda i: (0,)),
                    pl.BlockSpec((BLK, 1), lambda i: (i, 0))]
        args = (scout, bias, z)
        Cout = C + 1
    return pl.pallas_call(
        body,
        grid=(N // BLK,),
        in_specs=in_specs,
        out_specs=pl.BlockSpec((BLK, Cout), lambda i: (i, 0)),
        out_shape=jax.ShapeDtypeStruct((N, Cout), jnp.float32),
    )(*args)


def _stats(h):
    hr = h[:NREAL]
    return jnp.stack([jnp.mean(hr, 0), jnp.var(hr, 0)])


def _c_body(h_ref, st_ref, g_ref, bb_ref, wl_ref, wr_ref, ol_ref, or_ref, *,
            do_relu):
    st = st_ref[...]
    m = st[0]
    var = st[1]
    x = (h_ref[...] - m) * lax.rsqrt(var + 1e-5) * g_ref[...] + bb_ref[...]
    if do_relu:
        x = jnp.maximum(x, 0.0)
    ol_ref[...] = x @ wl_ref[...]
    or_ref[...] = x @ wr_ref[...]


def _c(h, st, g, bb, wl, wr, do_relu):
    N, C = h.shape
    M = wl.shape[1]
    full = lambda shp: pl.BlockSpec(shp, lambda i: (0,) * len(shp))
    ol, orr = pl.pallas_call(
        functools.partial(_c_body, do_relu=do_relu),
        grid=(N // BLK,),
        in_specs=[pl.BlockSpec((BLK, C), lambda i: (i, 0)),
                  full((2, C)), full((C,)), full((C,)),
                  full((C, M)), full((C, M))],
        out_specs=[pl.BlockSpec((BLK, M), lambda i: (i, 0)),
                   pl.BlockSpec((BLK, M), lambda i: (i, 0))],
        out_shape=[jax.ShapeDtypeStruct((N, M), jnp.float32),
                   jax.ShapeDtypeStruct((N, M), jnp.float32)],
    )(h, st, g, bb, wl, wr)
    return ol, orr


def _mlp_body(h_ref, w1_ref, b1_ref, w2_ref, b2_ref, w3_ref, b3_ref, o_ref):
    h = h_ref[...]
    h = jnp.maximum(h, 0.0)
    h = jnp.maximum(h @ w1_ref[...] + b1_ref[...], 0.0)
    h = jnp.maximum(h @ w2_ref[...] + b2_ref[...], 0.0)
    o_ref[...] = h @ w3_ref[...] + b3_ref[...]


def _pred_mlp(h, W1, b1, W2, b2, W3, b3):
    N = h.shape[0]
    full = lambda s: pl.BlockSpec(s, lambda i: (0,) * len(s))
    return pl.pallas_call(
        _mlp_body,
        grid=(N // BLK,),
        in_specs=[
            pl.BlockSpec((BLK, 64), lambda i: (i, 0)),
            full((64, 64)), full((64,)), full((64, 32)), full((32,)),
            full((32, 1)), full((1,)),
        ],
        out_specs=pl.BlockSpec((BLK, 1), lambda i: (i, 0)),
        out_shape=jax.ShapeDtypeStruct((N, 1), jnp.float32),
    )(h, W1, b1, W2, b2, W3, b3)


def kernel(x_low, x_high, z_std, ei_l2h, ei_hh, ds_Wl, ds_Wr, ds_att, ds_b, bn0_g, bn0_b, g1_Wl, g1_Wr, g1_att, g1_b, bn1_g, bn1_b, g2_Wl, g2_Wr, g2_att, g2_b, bn2_g, bn2_b, g3_Wl, g3_Wr, g3_att, g3_b, bn3_g, bn3_b, g4_Wl, g4_Wr, g4_att, g4_b, bn4_g, bn4_b, g5_Wl, g5_Wr, g5_att, g5_b, pred_W1, pred_b1, pred_W2, pred_b2, pred_W3, pred_b3):
    f32 = jnp.float32
    xl = jnp.pad(x_low.astype(f32), ((0, NTL - x_low.shape[0]), (0, 0)))
    xh = jnp.pad(x_high.astype(f32), ((0, NTH - NREAL), (0, 0)))
    z = jnp.pad(z_std.astype(f32), ((0, NTH - NREAL), (0, 0)))

    sL, dL, poffL = _prep_edges(ei_l2h[0], ei_l2h[1])
    loop = jnp.arange(NREAL, dtype=jnp.int32)
    sH0 = jnp.concatenate([ei_hh[0].astype(jnp.int32), loop])
    dH0 = jnp.concatenate([ei_hh[1].astype(jnp.int32), loop])
    sH, dH, poffH = _prep_edges(sH0, dH0)
    zeros80 = jnp.zeros((R + 1, 80), f32)
    zeros144 = jnp.zeros((R + 1, 144), f32)
    padsc = lambda s: jnp.pad(s, ((0, NTH - NB * R), (0, 0)))

    hl0 = _mm1(xl, ds_Wl)
    hr0 = _mm1(xh, ds_Wr)
    sc0 = _sc41(hl0, hr0, sL, dL, poffL, ds_att.reshape(-1), zeros80)[0]
    hraw = _a(padsc(sc0), ds_b, 1, z=z)
    cl, cr = _c(hraw, _stats(hraw), bn0_g, bn0_b, g1_Wl, g1_Wr, False)

    hh = [(g1_att, g1_b, bn1_g, bn1_b, g2_Wl, g2_Wr),
          (g2_att, g2_b, bn2_g, bn2_b, g3_Wl, g3_Wr),
          (g3_att, g3_b, bn3_g, bn3_b, g4_Wl, g4_Wr),
          (g4_att, g4_b, bn4_g, bn4_b, g5_Wl, g5_Wr)]
    for att, bias, bng, bnb, nwl, nwr in hh:
        sc = _sc82(cl, cr, sH, dH, poffH, att.reshape(-1), zeros144)[0]
        hraw = _a(padsc(sc), bias, 2)
        cl, cr = _c(hraw, _stats(hraw), bng, bnb, nwl, nwr, True)

    sc5 = _sc41(cl, cr, sH, dH, poffH, g5_att.reshape(-1), zeros80)[0]
    h5 = _a(padsc(sc5), g5_b, 1)
    out = _pred_mlp(h5, pred_W1, pred_b1, pred_W2, pred_b2, pred_W3, pred_b3)
    return out[:NREAL]
